# baseline (speedup 1.0000x reference)
import jax
import jax.numpy as jnp
from jax.experimental import pallas as pl
from jax.experimental.pallas import tpu as pltpu

_NUM_TYPE = 1000
_BATCH = 1024
_HIST = 200
_TN = 200
_GRID = _NUM_TYPE // _TN
_LANES = 128
_SUB = _BATCH // _LANES


def _body(ti_ref, tjs_ref, h_ref, emb_hbm, out_ref,
          gtT_ref, hf_ref, ebuf_ref, sems):
    i = pl.program_id(0)

    def _start(blk, slot):
        pltpu.make_async_copy(
            emb_hbm.at[pl.ds(blk * _TN, _TN), :], ebuf_ref.at[slot],
            sems.at[slot]).start()

    @pl.when(i == 0)
    def _init():
        _start(0, 0)
        gtT_ref[:] = jnp.exp(tjs_ref[0, :][:, None] - ti_ref[0, :][None, :])
        hf_ref[:] = h_ref[0].astype(jnp.float32)

    @pl.when(i + 1 < _GRID)
    def _prefetch():
        _start(i + 1, (i + 1) % 2)

    slot = i % 2
    pltpu.make_async_copy(
        emb_hbm.at[pl.ds(i * _TN, _TN), :], ebuf_ref.at[slot],
        sems.at[slot]).wait()

    P = jax.lax.dot_general(
        ebuf_ref[slot], hf_ref[:], (((1,), (1,)), ((), ())),
        preferred_element_type=jnp.float32)
    res = jnp.dot(P, gtT_ref[:], preferred_element_type=jnp.float32)
    out_ref[:] = res.reshape(_TN, _SUB, _LANES)


def kernel(ti, tjs, ci, cjs, emb_weight):
    del ci
    ti_row = jnp.reshape(ti, (1, _BATCH))
    out = pl.pallas_call(
        _body,
        grid=(_GRID,),
        in_specs=[
            pl.BlockSpec((1, _BATCH), lambda i: (0, 0)),
            pl.BlockSpec((1, _HIST), lambda i: (0, 0)),
            pl.BlockSpec((1, _HIST, _NUM_TYPE), lambda i: (0, 0, 0)),
            pl.BlockSpec(memory_space=pl.ANY),
        ],
        out_specs=pl.BlockSpec((_TN, _SUB, _LANES), lambda i: (i, 0, 0)),
        out_shape=jax.ShapeDtypeStruct((_NUM_TYPE, _SUB, _LANES), jnp.float32),
        scratch_shapes=[
            pltpu.VMEM((_HIST, _BATCH), jnp.float32),
            pltpu.VMEM((_HIST, _NUM_TYPE), jnp.float32),
            pltpu.VMEM((2, _TN, _NUM_TYPE), jnp.float32),
            pltpu.SemaphoreType.DMA((2,)),
        ],
    )(ti_row, tjs, cjs, emb_weight)
    return jnp.reshape(out, (_NUM_TYPE, _BATCH, 1))


# speedup vs baseline: 1.1689x; 1.1689x over previous
import jax
import jax.numpy as jnp
from jax.experimental import pallas as pl
from jax.experimental.pallas import tpu as pltpu

_NUM_TYPE = 1000
_BATCH = 1024
_HIST = 200
_TN = 200
_GRID = _NUM_TYPE // _TN
_LANES = 128
_SUB = _BATCH // _LANES


def _body(ti_ref, tjs_ref, h_ref, emb_hbm, out_ref,
          gtT_ref, hf_ref, ebuf_ref, sems):
    i = pl.program_id(0)

    def _start(blk, slot):
        pltpu.make_async_copy(
            emb_hbm.at[pl.ds(blk * _TN, _TN), :], ebuf_ref.at[slot],
            sems.at[slot]).start()

    @pl.when(i == 0)
    def _init():
        for blk in range(_GRID):
            _start(blk, blk)
        gtT_ref[:] = jnp.exp(tjs_ref[0, :][:, None] - ti_ref[0, :][None, :])
        hf_ref[:] = h_ref[0].astype(jnp.float32)

    slot = i
    pltpu.make_async_copy(
        emb_hbm.at[pl.ds(i * _TN, _TN), :], ebuf_ref.at[slot],
        sems.at[slot]).wait()

    P = jax.lax.dot_general(
        ebuf_ref[slot], hf_ref[:], (((1,), (1,)), ((), ())),
        preferred_element_type=jnp.float32)
    res = jnp.dot(P, gtT_ref[:], preferred_element_type=jnp.float32)
    out_ref[:] = res.reshape(_TN, _SUB, _LANES)


def kernel(ti, tjs, ci, cjs, emb_weight):
    del ci
    ti_row = jnp.reshape(ti, (1, _BATCH))
    out = pl.pallas_call(
        _body,
        grid=(_GRID,),
        in_specs=[
            pl.BlockSpec((1, _BATCH), lambda i: (0, 0)),
            pl.BlockSpec((1, _HIST), lambda i: (0, 0)),
            pl.BlockSpec((1, _HIST, _NUM_TYPE), lambda i: (0, 0, 0)),
            pl.BlockSpec(memory_space=pl.ANY),
        ],
        out_specs=pl.BlockSpec((_TN, _SUB, _LANES), lambda i: (i, 0, 0)),
        out_shape=jax.ShapeDtypeStruct((_NUM_TYPE, _SUB, _LANES), jnp.float32),
        scratch_shapes=[
            pltpu.VMEM((_HIST, _BATCH), jnp.float32),
            pltpu.VMEM((_HIST, _NUM_TYPE), jnp.float32),
            pltpu.VMEM((_GRID, _TN, _NUM_TYPE), jnp.float32),
            pltpu.SemaphoreType.DMA((_GRID,)),
        ],
    )(ti_row, tjs, cjs, emb_weight)
    return jnp.reshape(out, (_NUM_TYPE, _BATCH, 1))
